# Initial kernel scaffold; baseline (speedup 1.0000x reference)
import jax
import jax.numpy as jnp
from jax.experimental import pallas as pl


def kernel(x, edge_attr, edge_index, Wn, bn, Wed, bed, We, be, Wu, bu, Wd, bd):
    raise NotImplementedError("write your pallas kernel here")


# speedup vs baseline: 4.0514x; 4.0514x over previous
import functools

import jax
import jax.numpy as jnp
from jax import lax
from jax.experimental import pallas as pl
from jax.experimental.pallas import tpu as pltpu
from jax.experimental.pallas import tpu_sc as plsc

N = 50000
EMB = 64
NHALF = 25000
ACC = 25088
TRASH = 25080
NTILES = 16
ROWS_PER_TILE = ACC // NTILES
CH = 128
GROUP = 8
NGROUPS = 49
CPT = GROUP * NGROUPS
E_PAD = NTILES * CPT * CH
CH8 = 512
CPT8 = E_PAD // (NTILES * CH8)
BLK = 2000

_mesh = plsc.VectorSubcoreMesh(core_axis_name="c", subcore_axis_name="s")


def _sc_segsum64(h, src_pad, dst_pad, zrows):
    @functools.partial(
        pl.kernel,
        mesh=_mesh,
        out_type=jax.ShapeDtypeStruct((2 * ACC, EMB), jnp.float32),
        compiler_params=pltpu.CompilerParams(use_tc_tiling_on_sc=False),
        scratch_types=[
            pltpu.VMEM_SHARED((ACC, EMB), jnp.float32),
            pltpu.VMEM((GROUP * CH,), jnp.int32),
            pltpu.VMEM((GROUP * CH,), jnp.int32),
            pltpu.VMEM((2, CH), jnp.int32),
            pltpu.VMEM((2, CH, EMB), jnp.float32),
            pltpu.SemaphoreType.DMA,
            pltpu.SemaphoreType.DMA,
        ],
    )
    def k(h_hbm, src_hbm, dst_hbm, z_hbm, out_hbm,
          acc, srcbuf, dstbuf, sidx, rows, sem0, sem1):
        c = lax.axis_index("c")
        t = lax.axis_index("s")
        base = c * NHALF

        pltpu.sync_copy(z_hbm, acc.at[pl.ds(t * ROWS_PER_TILE, ROWS_PER_TILE)])
        plsc.subcore_barrier()

        tile_off = t * CPT * CH

        def group_body(g, carry):
            goff = tile_off + g * (GROUP * CH)
            pltpu.sync_copy(src_hbm.at[pl.ds(goff, GROUP * CH)], srcbuf)
            pltpu.sync_copy(dst_hbm.at[pl.ds(goff, GROUP * CH)], dstbuf)
            cp = pltpu.async_copy(
                h_hbm.at[srcbuf.at[pl.ds(0, CH)]], rows.at[0], sem0)
            for j in range(GROUP):
                nxt = None
                if j + 1 < GROUP:
                    nxt = pltpu.async_copy(
                        h_hbm.at[srcbuf.at[pl.ds((j + 1) * CH, CH)]],
                        rows.at[(j + 1) % 2],
                        sem1 if (j + 1) % 2 else sem0)
                def remap(i, u, _j=j):
                    d = dstbuf[pl.ds(_j * CH + i * 16, 16)]
                    ok = (d >= base) & (d < base + NHALF)
                    sidx[_j % 2, pl.ds(i * 16, 16)] = jnp.where(
                        ok, d - base, TRASH)
                    return u
                lax.fori_loop(0, CH // 16, remap, 0)
                cp.wait()
                pltpu.sync_copy(rows.at[j % 2],
                                acc.at[sidx.at[j % 2]], add=True)
                cp = nxt
            return carry

        lax.fori_loop(0, NGROUPS, group_body, 0)
        plsc.subcore_barrier()
        pltpu.sync_copy(
            acc.at[pl.ds(t * ROWS_PER_TILE, ROWS_PER_TILE)],
            out_hbm.at[pl.ds(c * ACC + t * ROWS_PER_TILE, ROWS_PER_TILE)])

    return k(h, src_pad, dst_pad, zrows)


def _sc_segsum8(ea8_pad, dst_pad, zrows):
    @functools.partial(
        pl.kernel,
        mesh=_mesh,
        out_type=jax.ShapeDtypeStruct((2 * ACC, 8), jnp.float32),
        compiler_params=pltpu.CompilerParams(use_tc_tiling_on_sc=False),
        scratch_types=[
            pltpu.VMEM_SHARED((ACC, 8), jnp.float32),
            pltpu.VMEM((CH8,), jnp.int32),
            pltpu.VMEM((4, 128), jnp.int32),
            pltpu.VMEM((CH8, 8), jnp.float32),
        ],
    )
    def k(ea_hbm, dst_hbm, z_hbm, out_hbm, acc, dstbuf, sidx, rows):
        c = lax.axis_index("c")
        t = lax.axis_index("s")
        base = c * NHALF

        pltpu.sync_copy(z_hbm, acc.at[pl.ds(t * ROWS_PER_TILE, ROWS_PER_TILE)])
        plsc.subcore_barrier()

        tile_off = t * CPT8 * CH8

        def chunk_body(jj, carry):
            off = tile_off + jj * CH8
            pltpu.sync_copy(dst_hbm.at[pl.ds(off, CH8)], dstbuf)
            pltpu.sync_copy(ea_hbm.at[pl.ds(off, CH8)], rows)
            for q in range(4):
                def remap(i, u, _q=q):
                    d = dstbuf[pl.ds(_q * 128 + i * 16, 16)]
                    ok = (d >= base) & (d < base + NHALF)
                    sidx[_q, pl.ds(i * 16, 16)] = jnp.where(ok, d - base, TRASH)
                    return u
                lax.fori_loop(0, 8, remap, 0)
                pltpu.sync_copy(rows.at[pl.ds(q * 128, 128)],
                                acc.at[sidx.at[q]], add=True)
            return carry

        lax.fori_loop(0, CPT8, chunk_body, 0)
        plsc.subcore_barrier()
        pltpu.sync_copy(
            acc.at[pl.ds(t * ROWS_PER_TILE, ROWS_PER_TILE)],
            out_hbm.at[pl.ds(c * ACC + t * ROWS_PER_TILE, ROWS_PER_TILE)])

    return k(ea8_pad, dst_pad, zrows)


def _unpad(p_pad):
    return jnp.concatenate([p_pad[:NHALF], p_pad[ACC:ACC + NHALF]], axis=0)


def _mm(a, b):
    return jnp.dot(a, b, preferred_element_type=jnp.float32)


def _embed(x8, Wn8, bn8):
    def body(x_ref, w_ref, b_ref, o_ref):
        o_ref[...] = jnp.maximum(
            _mm(x_ref[...], w_ref[...]) + b_ref[0:1, :], 0.0)

    return pl.pallas_call(
        body,
        grid=(N // BLK,),
        in_specs=[pl.BlockSpec((BLK, 8), lambda i: (i, 0)),
                  pl.BlockSpec((8, EMB), lambda i: (0, 0)),
                  pl.BlockSpec((8, EMB), lambda i: (0, 0))],
        out_specs=pl.BlockSpec((BLK, EMB), lambda i: (i, 0)),
        out_shape=jax.ShapeDtypeStruct((N, EMB), jnp.float32),
    )(x8, Wn8, bn8)


def _update(h, p, seacnt, Wed8, We_i, be_i8, Wu_i, bu_i8):
    def body(h_ref, p_ref, sc_ref, wed_ref, we_ref, be_ref, wu_ref, bu_ref,
             o_ref):
        sc8 = sc_ref[...]
        cnt = sc8[:, 4:5]
        sea = _mm(sc8, wed_ref[...])
        agg = (_mm(p_ref[...], we_ref[0:EMB, :])
               + _mm(sea, we_ref[EMB:2 * EMB, :])
               + cnt * be_ref[0:1, :]) / jnp.maximum(cnt, 1.0)
        o = (_mm(h_ref[...], wu_ref[0:EMB, :])
             + _mm(agg, wu_ref[EMB:2 * EMB, :]) + bu_ref[0:1, :])
        o_ref[...] = jnp.maximum(o, 0.0)

    return pl.pallas_call(
        body,
        grid=(N // BLK,),
        in_specs=[pl.BlockSpec((BLK, EMB), lambda i: (i, 0)),
                  pl.BlockSpec((BLK, EMB), lambda i: (i, 0)),
                  pl.BlockSpec((BLK, 8), lambda i: (i, 0)),
                  pl.BlockSpec((8, EMB), lambda i: (0, 0)),
                  pl.BlockSpec((2 * EMB, EMB), lambda i: (0, 0)),
                  pl.BlockSpec((8, EMB), lambda i: (0, 0)),
                  pl.BlockSpec((2 * EMB, EMB), lambda i: (0, 0)),
                  pl.BlockSpec((8, EMB), lambda i: (0, 0))],
        out_specs=pl.BlockSpec((BLK, EMB), lambda i: (i, 0)),
        out_shape=jax.ShapeDtypeStruct((N, EMB), jnp.float32),
    )(h, p, seacnt, Wed8, We_i, be_i8, Wu_i, bu_i8)


def _decode(h, Wd8, bd8):
    def body(h_ref, w_ref, b_ref, o_ref):
        o_ref[...] = _mm(h_ref[...], w_ref[...]) + b_ref[0:1, :]

    return pl.pallas_call(
        body,
        grid=(N // BLK,),
        in_specs=[pl.BlockSpec((BLK, EMB), lambda i: (i, 0)),
                  pl.BlockSpec((EMB, 8), lambda i: (0, 0)),
                  pl.BlockSpec((8, 8), lambda i: (0, 0))],
        out_specs=pl.BlockSpec((BLK, 8), lambda i: (i, 0)),
        out_shape=jax.ShapeDtypeStruct((N, 8), jnp.float32),
    )(h, Wd8, bd8)


def kernel(x, edge_attr, edge_index, Wn, bn, Wed, bed, We, be, Wu, bu, Wd, bd):
    E = edge_index.shape[1]
    src = edge_index[0]
    dst = edge_index[1]

    src_pad = jnp.concatenate(
        [src, jnp.zeros((E_PAD - E,), jnp.int32)])
    dst_pad = jnp.concatenate(
        [dst, jnp.full((E_PAD - E,), -1, jnp.int32)])
    ea8_pad = jnp.concatenate([
        jnp.concatenate([edge_attr,
                         jnp.ones((E, 1), jnp.float32),
                         jnp.zeros((E, 3), jnp.float32)], axis=1),
        jnp.zeros((E_PAD - E, 8), jnp.float32)], axis=0)

    x8 = jnp.concatenate([x, jnp.zeros((N, 1), jnp.float32)], axis=1)
    Wn8 = jnp.concatenate([Wn, jnp.zeros((1, EMB), jnp.float32)], axis=0)
    bn8 = jnp.broadcast_to(bn[None, :], (8, EMB))
    Wed8 = jnp.concatenate(
        [Wed, bed[None, :], jnp.zeros((3, EMB), jnp.float32)], axis=0)
    Wd8 = jnp.concatenate([Wd, jnp.zeros((EMB, 5), jnp.float32)], axis=1)
    bd8 = jnp.broadcast_to(
        jnp.concatenate([bd, jnp.zeros((5,), jnp.float32)])[None, :], (8, 8))

    z64 = jnp.zeros((ROWS_PER_TILE, EMB), jnp.float32)
    z8 = jnp.zeros((ROWS_PER_TILE, 8), jnp.float32)

    seacnt = _unpad(_sc_segsum8(ea8_pad, dst_pad, z8))

    h = _embed(x8, Wn8, bn8)
    for i in range(6):
        p = _unpad(_sc_segsum64(h, src_pad, dst_pad, z64))
        h = _update(h, p, seacnt, Wed8, We[i],
                    jnp.broadcast_to(be[i][None, :], (8, EMB)),
                    Wu[i],
                    jnp.broadcast_to(bu[i][None, :], (8, EMB)))
    return _decode(h, Wd8, bd8)[:, :3]


# speedup vs baseline: 5.8189x; 1.4363x over previous
import functools

import jax
import jax.numpy as jnp
from jax import lax
from jax.experimental import pallas as pl
from jax.experimental.pallas import tpu as pltpu
from jax.experimental.pallas import tpu_sc as plsc

N = 50000
EMB = 64
HW = 32
NPAD = 52000
TRASH = 51968
NTILES = 16
RPT = NPAD // NTILES
CH = 128
GROUP = 8
NGROUPS = 49
CPT = GROUP * NGROUPS
E_PAD = NTILES * CPT * CH
CH8 = 512
EHALF = E_PAD // 2
CPT8 = EHALF // (NTILES * CH8)
BLK = 2000
NBLK = NPAD // BLK

_mesh = plsc.VectorSubcoreMesh(core_axis_name="c", subcore_axis_name="s")


def _sc_segsum64(table, srcidx, dst2d, zrows):
    @functools.partial(
        pl.kernel,
        mesh=_mesh,
        out_type=jax.ShapeDtypeStruct((2 * NPAD, HW), jnp.float32),
        compiler_params=pltpu.CompilerParams(use_tc_tiling_on_sc=False),
        scratch_types=[
            pltpu.VMEM_SHARED((NPAD, HW), jnp.float32),
            pltpu.VMEM((GROUP * CH,), jnp.int32),
            pltpu.VMEM((GROUP, CH), jnp.int32),
            pltpu.VMEM((2, CH, HW), jnp.float32),
            pltpu.SemaphoreType.DMA,
            pltpu.SemaphoreType.DMA,
        ],
    )
    def k(tab_hbm, src_hbm, dst_hbm, z_hbm, out_hbm,
          acc, gsrc, gdst, rows, sem0, sem1):
        c = lax.axis_index("c")
        t = lax.axis_index("s")

        pltpu.sync_copy(z_hbm, acc.at[pl.ds(t * RPT, RPT)])
        plsc.subcore_barrier()

        tile_edge = t * CPT * CH
        tile_drow = t * CPT

        def group_body(g, carry):
            eoff = tile_edge + g * (GROUP * CH)
            pltpu.sync_copy(src_hbm.at[pl.ds(c * E_PAD + eoff, GROUP * CH)],
                            gsrc)
            pltpu.sync_copy(dst_hbm.at[pl.ds(tile_drow + g * GROUP, GROUP)],
                            gdst)
            cp = pltpu.async_copy(
                tab_hbm.at[gsrc.at[pl.ds(0, CH)]], rows.at[0], sem0)
            for j in range(GROUP):
                nxt = None
                if j + 1 < GROUP:
                    nxt = pltpu.async_copy(
                        tab_hbm.at[gsrc.at[pl.ds((j + 1) * CH, CH)]],
                        rows.at[(j + 1) % 2],
                        sem1 if (j + 1) % 2 else sem0)
                cp.wait()
                pltpu.sync_copy(rows.at[j % 2],
                                acc.at[gdst.at[j]], add=True)
                cp = nxt
            return carry

        lax.fori_loop(0, NGROUPS, group_body, 0)
        plsc.subcore_barrier()
        pltpu.sync_copy(acc.at[pl.ds(t * RPT, RPT)],
                        out_hbm.at[pl.ds(c * NPAD + t * RPT, RPT)])

    return k(table, srcidx, dst2d, zrows)


def _sc_segsum8(ea8_pad, dst2d, zrows):
    @functools.partial(
        pl.kernel,
        mesh=_mesh,
        out_type=jax.ShapeDtypeStruct((2 * NPAD, 8), jnp.float32),
        compiler_params=pltpu.CompilerParams(use_tc_tiling_on_sc=False),
        scratch_types=[
            pltpu.VMEM_SHARED((NPAD, 8), jnp.float32),
            pltpu.VMEM((4, 128), jnp.int32),
            pltpu.VMEM((CH8, 8), jnp.float32),
        ],
    )
    def k(ea_hbm, dst_hbm, z_hbm, out_hbm, acc, gdst, rows):
        c = lax.axis_index("c")
        t = lax.axis_index("s")

        pltpu.sync_copy(z_hbm, acc.at[pl.ds(t * RPT, RPT)])
        plsc.subcore_barrier()

        tile_edge = c * EHALF + t * CPT8 * CH8
        tile_drow = tile_edge // 128

        def chunk_body(jj, carry):
            pltpu.sync_copy(ea_hbm.at[pl.ds(tile_edge + jj * CH8, CH8)], rows)
            pltpu.sync_copy(dst_hbm.at[pl.ds(tile_drow + jj * 4, 4)], gdst)
            for q in range(4):
                pltpu.sync_copy(rows.at[pl.ds(q * 128, 128)],
                                acc.at[gdst.at[q]], add=True)
            return carry

        lax.fori_loop(0, CPT8, chunk_body, 0)
        plsc.subcore_barrier()
        pltpu.sync_copy(acc.at[pl.ds(t * RPT, RPT)],
                        out_hbm.at[pl.ds(c * NPAD + t * RPT, RPT)])

    return k(ea8_pad, dst2d, zrows)


def _mm(a, b):
    return jnp.dot(a, b, preferred_element_type=jnp.float32)


def _half_specs(i_lo, i_hi, w):
    return [pl.BlockSpec((BLK, w), i_lo), pl.BlockSpec((BLK, w), i_hi)]


def _embed(x8, Wn8, bn8):
    def body(x_ref, w_ref, b_ref, lo_ref, hi_ref):
        r = jnp.maximum(_mm(x_ref[...], w_ref[...]) + b_ref[0:1, :], 0.0)
        lo_ref[...] = r[:, 0:HW]
        hi_ref[...] = r[:, HW:EMB]

    return pl.pallas_call(
        body,
        grid=(N // BLK,),
        in_specs=[pl.BlockSpec((BLK, 8), lambda i: (i, 0)),
                  pl.BlockSpec((8, EMB), lambda i: (0, 0)),
                  pl.BlockSpec((8, EMB), lambda i: (0, 0))],
        out_specs=[pl.BlockSpec((BLK, HW), lambda i: (i, 0)),
                   pl.BlockSpec((BLK, HW), lambda i: (i, 0))],
        out_shape=[jax.ShapeDtypeStruct((N, HW), jnp.float32),
                   jax.ShapeDtypeStruct((N, HW), jnp.float32)],
    )(x8, Wn8, bn8)


def _update(h_lo, h_hi, p2, sc2, Wed8, We_i, be_i8, Wu_i, bu_i8):
    def body(hl_ref, hh_ref, pl_ref, ph_ref, sl_ref, sh_ref,
             wed_ref, we_ref, be_ref, wu_ref, bu_ref, lo_ref, hi_ref):
        sc8 = sl_ref[...] + sh_ref[...]
        cnt = sc8[:, 4:5]
        sea = _mm(sc8, wed_ref[...])
        agg = (_mm(pl_ref[...], we_ref[0:HW, :])
               + _mm(ph_ref[...], we_ref[HW:EMB, :])
               + _mm(sea, we_ref[EMB:2 * EMB, :])
               + cnt * be_ref[0:1, :]) / jnp.maximum(cnt, 1.0)
        o = (_mm(hl_ref[...], wu_ref[0:HW, :])
             + _mm(hh_ref[...], wu_ref[HW:EMB, :])
             + _mm(agg, wu_ref[EMB:2 * EMB, :]) + bu_ref[0:1, :])
        o = jnp.maximum(o, 0.0)
        lo_ref[...] = o[:, 0:HW]
        hi_ref[...] = o[:, HW:EMB]

    lo_map = lambda i: (i, 0)
    hi_map = lambda i: (i + NBLK, 0)
    return pl.pallas_call(
        body,
        grid=(N // BLK,),
        in_specs=([pl.BlockSpec((BLK, HW), lo_map),
                   pl.BlockSpec((BLK, HW), lo_map)]
                  + _half_specs(lo_map, hi_map, HW)
                  + _half_specs(lo_map, hi_map, 8)
                  + [pl.BlockSpec((8, EMB), lambda i: (0, 0)),
                     pl.BlockSpec((2 * EMB, EMB), lambda i: (0, 0)),
                     pl.BlockSpec((8, EMB), lambda i: (0, 0)),
                     pl.BlockSpec((2 * EMB, EMB), lambda i: (0, 0)),
                     pl.BlockSpec((8, EMB), lambda i: (0, 0))]),
        out_specs=[pl.BlockSpec((BLK, HW), lo_map),
                   pl.BlockSpec((BLK, HW), lo_map)],
        out_shape=[jax.ShapeDtypeStruct((N, HW), jnp.float32),
                   jax.ShapeDtypeStruct((N, HW), jnp.float32)],
    )(h_lo, h_hi, p2, p2, sc2, sc2, Wed8, We_i, be_i8, Wu_i, bu_i8)


def _decode(h_lo, h_hi, Wd8, bd8):
    def body(hl_ref, hh_ref, w_ref, b_ref, o_ref):
        o_ref[...] = (_mm(hl_ref[...], w_ref[0:HW, :])
                      + _mm(hh_ref[...], w_ref[HW:EMB, :]) + b_ref[0:1, :])

    return pl.pallas_call(
        body,
        grid=(N // BLK,),
        in_specs=[pl.BlockSpec((BLK, HW), lambda i: (i, 0)),
                  pl.BlockSpec((BLK, HW), lambda i: (i, 0)),
                  pl.BlockSpec((EMB, 8), lambda i: (0, 0)),
                  pl.BlockSpec((8, 8), lambda i: (0, 0))],
        out_specs=pl.BlockSpec((BLK, 8), lambda i: (i, 0)),
        out_shape=jax.ShapeDtypeStruct((N, 8), jnp.float32),
    )(h_lo, h_hi, Wd8, bd8)


def kernel(x, edge_attr, edge_index, Wn, bn, Wed, bed, We, be, Wu, bu, Wd, bd):
    E = edge_index.shape[1]
    src = edge_index[0]
    dst = edge_index[1]

    src_pad = jnp.concatenate([src, jnp.zeros((E_PAD - E,), jnp.int32)])
    srcidx = jnp.concatenate([src_pad, src_pad + N])
    dst2d = jnp.concatenate(
        [dst, jnp.full((E_PAD - E,), TRASH, jnp.int32)]).reshape(-1, 128)
    ea8_pad = jnp.concatenate([
        jnp.concatenate([edge_attr,
                         jnp.ones((E, 1), jnp.float32),
                         jnp.zeros((E, 3), jnp.float32)], axis=1),
        jnp.zeros((E_PAD - E, 8), jnp.float32)], axis=0)

    x8 = jnp.concatenate([x, jnp.zeros((N, 1), jnp.float32)], axis=1)
    Wn8 = jnp.concatenate([Wn, jnp.zeros((1, EMB), jnp.float32)], axis=0)
    bn8 = jnp.broadcast_to(bn[None, :], (8, EMB))
    Wed8 = jnp.concatenate(
        [Wed, bed[None, :], jnp.zeros((3, EMB), jnp.float32)], axis=0)
    Wd8 = jnp.concatenate([Wd, jnp.zeros((EMB, 5), jnp.float32)], axis=1)
    bd8 = jnp.broadcast_to(
        jnp.concatenate([bd, jnp.zeros((5,), jnp.float32)])[None, :], (8, 8))

    z32 = jnp.zeros((RPT, HW), jnp.float32)
    z8 = jnp.zeros((RPT, 8), jnp.float32)

    sc2 = _sc_segsum8(ea8_pad, dst2d, z8)

    h_lo, h_hi = _embed(x8, Wn8, bn8)
    for i in range(6):
        table = jnp.concatenate([h_lo, h_hi], axis=0)
        p2 = _sc_segsum64(table, srcidx, dst2d, z32)
        h_lo, h_hi = _update(h_lo, h_hi, p2, sc2, Wed8, We[i],
                             jnp.broadcast_to(be[i][None, :], (8, EMB)),
                             Wu[i],
                             jnp.broadcast_to(bu[i][None, :], (8, EMB)))
    return _decode(h_lo, h_hi, Wd8, bd8)[:, :3]


# speedup vs baseline: 8.0298x; 1.3800x over previous
import functools

import jax
import jax.numpy as jnp
from jax import lax
from jax.experimental import pallas as pl
from jax.experimental.pallas import tpu as pltpu
from jax.experimental.pallas import tpu_sc as plsc

N = 50000
EMB = 64
HW = 32
NP = 51200
NPAD = 54400
TRASH = 54000
NTILES = 16
RPT = NPAD // NTILES
CH = 128
GROUP = 8
NGROUPS = 49
CPT = GROUP * NGROUPS
E_PAD = NTILES * CPT * CH
EHALF = E_PAD // 2
G8 = 7
NG8 = EHALF // (NTILES * CH * G8)
BLK = 3200

_mesh = plsc.VectorSubcoreMesh(core_axis_name="c", subcore_axis_name="s")


def _sc_segsum64(table, srcidx, dst2d, zrows):
    @functools.partial(
        pl.kernel,
        mesh=_mesh,
        out_type=jax.ShapeDtypeStruct((NPAD, 128), jnp.float32),
        compiler_params=pltpu.CompilerParams(use_tc_tiling_on_sc=False, needs_layout_passes=False),
        scratch_types=[
            pltpu.VMEM_SHARED((NPAD, HW), jnp.float32),
            pltpu.VMEM((GROUP * CH,), jnp.int32),
            pltpu.VMEM((GROUP, CH), jnp.int32),
            pltpu.VMEM((2, CH, HW), jnp.float32),
            pltpu.SemaphoreType.DMA,
            pltpu.SemaphoreType.DMA,
        ],
    )
    def k(tab_hbm, src_hbm, dst_hbm, z_hbm, out_hbm,
          acc, gsrc, gdst, rows, sem0, sem1):
        c = lax.axis_index("c")
        t = lax.axis_index("s")

        pltpu.sync_copy(z_hbm, acc.at[pl.ds(t * RPT, RPT)])
        plsc.subcore_barrier()

        tile_edge = t * CPT * CH
        tile_drow = t * CPT

        def group_body(g, carry):
            eoff = tile_edge + g * (GROUP * CH)
            pltpu.sync_copy(src_hbm.at[pl.ds(c * E_PAD + eoff, GROUP * CH)],
                            gsrc)
            pltpu.sync_copy(dst_hbm.at[pl.ds(tile_drow + g * GROUP, GROUP)],
                            gdst)
            cp = pltpu.async_copy(
                tab_hbm.at[gsrc.at[pl.ds(0, CH)]], rows.at[0], sem0)
            for j in range(GROUP):
                nxt = None
                if j + 1 < GROUP:
                    nxt = pltpu.async_copy(
                        tab_hbm.at[gsrc.at[pl.ds((j + 1) * CH, CH)]],
                        rows.at[(j + 1) % 2],
                        sem1 if (j + 1) % 2 else sem0)
                cp.wait()
                pltpu.sync_copy(rows.at[j % 2],
                                acc.at[gdst.at[j]], add=True)
                cp = nxt
            return carry

        lax.fori_loop(0, NGROUPS, group_body, 0)
        plsc.subcore_barrier()
        pltpu.sync_copy(acc.at[pl.ds(t * RPT, RPT)],
                        out_hbm.at[pl.ds(t * RPT, RPT), pl.ds(c * HW, HW)])

    return k(table, srcidx, dst2d, zrows)


def _sc_segsum8(c0, c1, c2, c3, dst2d, zrows):
    @functools.partial(
        pl.kernel,
        mesh=_mesh,
        out_type=jax.ShapeDtypeStruct((NPAD, 128), jnp.float32),
        compiler_params=pltpu.CompilerParams(use_tc_tiling_on_sc=False, needs_layout_passes=False),
        scratch_types=[
            pltpu.VMEM_SHARED((NPAD, 8), jnp.float32),
            pltpu.VMEM((4, G8 * CH), jnp.float32),
            pltpu.VMEM((G8, CH), jnp.int32),
            pltpu.VMEM((CH, 8), jnp.float32),
        ],
    )
    def k(c0_hbm, c1_hbm, c2_hbm, c3_hbm, dst_hbm, z_hbm, out_hbm,
          acc, cols, gdst, rows8):
        c = lax.axis_index("c")
        t = lax.axis_index("s")

        pltpu.sync_copy(z_hbm, acc.at[pl.ds(t * RPT, RPT)])

        lanes = lax.iota(jnp.int32, 16)

        def init_body(kk, carry):
            f = kk * 16 + lanes
            v = jnp.where((f & 7) == 4, 1.0, 0.0).astype(jnp.float32)
            plsc.store_scatter(rows8, [f >> 3, f & 7], v)
            return carry

        lax.fori_loop(0, CH * 8 // 16, init_body, 0)
        plsc.subcore_barrier()

        tile_edge = c * EHALF + t * NG8 * G8 * CH
        tile_drow = tile_edge // CH
        chbms = [c0_hbm, c1_hbm, c2_hbm, c3_hbm]

        def group_body(g, carry):
            eoff = tile_edge + g * (G8 * CH)
            for ci in range(4):
                pltpu.sync_copy(chbms[ci].at[pl.ds(eoff, G8 * CH)],
                                cols.at[ci])
            pltpu.sync_copy(dst_hbm.at[pl.ds(tile_drow + g * G8, G8)], gdst)
            for j in range(G8):
                def ileave(q, carry2, _j=j):
                    e = q * 16 + lanes
                    for ci in range(4):
                        v = cols[ci, pl.ds(_j * CH + q * 16, 16)]
                        plsc.store_scatter(
                            rows8, [e, jnp.full((16,), ci, jnp.int32)], v)
                    return carry2
                lax.fori_loop(0, CH // 16, ileave, 0)
                pltpu.sync_copy(rows8, acc.at[gdst.at[j]], add=True)
            return carry

        lax.fori_loop(0, NG8, group_body, 0)
        plsc.subcore_barrier()
        pltpu.sync_copy(acc.at[pl.ds(t * RPT, RPT)],
                        out_hbm.at[pl.ds(t * RPT, RPT), pl.ds(c * 8, 8)])

    return k(c0, c1, c2, c3, dst2d, zrows)


def _mm(a, b):
    return jnp.dot(a, b, preferred_element_type=jnp.float32)


_W128 = lambda i: (i, 0)
_W0 = lambda i: (0, 0)


def _embed(x8, Wn8, bn8):
    def body(x_ref, w_ref, b_ref, o_ref):
        r = jnp.maximum(_mm(x_ref[...], w_ref[...]) + b_ref[0:1, :], 0.0)
        o_ref[...] = jnp.concatenate(
            [r, jnp.zeros((BLK, 128 - EMB), jnp.float32)], axis=1)

    return pl.pallas_call(
        body,
        grid=(NP // BLK,),
        in_specs=[pl.BlockSpec((BLK, 8), _W128),
                  pl.BlockSpec((8, EMB), _W0),
                  pl.BlockSpec((8, EMB), _W0)],
        out_specs=pl.BlockSpec((BLK, 128), _W128),
        out_shape=jax.ShapeDtypeStruct((NP, 128), jnp.float32),
    )(x8, Wn8, bn8)


def _update(h128, p128, s128, Wed8, We_i, be_i8, Wu_i, bu_i8):
    def body(h_ref, p_ref, s_ref, wed_ref, we_ref, be_ref, wu_ref, bu_ref,
             o_ref):
        sc8 = s_ref[:, 0:8] + s_ref[:, 8:16]
        cnt = sc8[:, 4:5]
        sea = _mm(sc8, wed_ref[...])
        agg = (_mm(p_ref[:, 0:EMB], we_ref[0:EMB, :])
               + _mm(sea, we_ref[EMB:2 * EMB, :])
               + cnt * be_ref[0:1, :]) / jnp.maximum(cnt, 1.0)
        o = (_mm(h_ref[:, 0:EMB], wu_ref[0:EMB, :])
             + _mm(agg, wu_ref[EMB:2 * EMB, :]) + bu_ref[0:1, :])
        o = jnp.maximum(o, 0.0)
        o_ref[...] = jnp.concatenate(
            [o, jnp.zeros((BLK, 128 - EMB), jnp.float32)], axis=1)

    return pl.pallas_call(
        body,
        grid=(NP // BLK,),
        in_specs=[pl.BlockSpec((BLK, 128), _W128),
                  pl.BlockSpec((BLK, 128), _W128),
                  pl.BlockSpec((BLK, 128), _W128),
                  pl.BlockSpec((8, EMB), _W0),
                  pl.BlockSpec((2 * EMB, EMB), _W0),
                  pl.BlockSpec((8, EMB), _W0),
                  pl.BlockSpec((2 * EMB, EMB), _W0),
                  pl.BlockSpec((8, EMB), _W0)],
        out_specs=pl.BlockSpec((BLK, 128), _W128),
        out_shape=jax.ShapeDtypeStruct((NP, 128), jnp.float32),
    )(h128, p128, s128, Wed8, We_i, be_i8, Wu_i, bu_i8)


def _decode(h128, Wd8, bd8):
    def body(h_ref, w_ref, b_ref, o_ref):
        o_ref[...] = _mm(h_ref[:, 0:EMB], w_ref[...]) + b_ref[0:1, :]

    return pl.pallas_call(
        body,
        grid=(NP // BLK,),
        in_specs=[pl.BlockSpec((BLK, 128), _W128),
                  pl.BlockSpec((EMB, 8), _W0),
                  pl.BlockSpec((8, 8), _W0)],
        out_specs=pl.BlockSpec((BLK, 8), _W128),
        out_shape=jax.ShapeDtypeStruct((NP, 8), jnp.float32),
    )(h128, Wd8, bd8)


def kernel(x, edge_attr, edge_index, Wn, bn, Wed, bed, We, be, Wu, bu, Wd, bd):
    E = edge_index.shape[1]
    src = edge_index[0]
    dst = edge_index[1]

    src_pad = jnp.concatenate([src, jnp.zeros((E_PAD - E,), jnp.int32)])
    srcidx = jnp.concatenate([4 * src_pad, 4 * src_pad + 1])
    dst2d = jnp.concatenate(
        [dst, jnp.full((E_PAD - E,), TRASH, jnp.int32)]).reshape(-1, CH)
    zpad = jnp.zeros((E_PAD - E,), jnp.float32)
    ecols = [jnp.concatenate([edge_attr[:, i], zpad]) for i in range(4)]

    x8 = jnp.pad(x, ((0, NP - N), (0, 1)))
    Wn8 = jnp.concatenate([Wn, jnp.zeros((1, EMB), jnp.float32)], axis=0)
    bn8 = jnp.broadcast_to(bn[None, :], (8, EMB))
    Wed8 = jnp.concatenate(
        [Wed, bed[None, :], jnp.zeros((3, EMB), jnp.float32)], axis=0)
    Wd8 = jnp.concatenate([Wd, jnp.zeros((EMB, 5), jnp.float32)], axis=1)
    bd8 = jnp.broadcast_to(
        jnp.concatenate([bd, jnp.zeros((5,), jnp.float32)])[None, :], (8, 8))

    z32 = jnp.zeros((RPT, HW), jnp.float32)
    z8 = jnp.zeros((RPT, 8), jnp.float32)

    s128 = _sc_segsum8(*ecols, dst2d, z8)

    h128 = _embed(x8, Wn8, bn8)
    for i in range(6):
        table = h128.reshape(4 * NP, HW)
        p128 = _sc_segsum64(table, srcidx, dst2d, z32)
        h128 = _update(h128, p128, s128, Wed8, We[i],
                       jnp.broadcast_to(be[i][None, :], (8, EMB)),
                       Wu[i],
                       jnp.broadcast_to(bu[i][None, :], (8, EMB)))
    return _decode(h128, Wd8, bd8)[:N, :3]


# speedup vs baseline: 9.1346x; 1.1376x over previous
import functools

import jax
import jax.numpy as jnp
from jax import lax
from jax.experimental import pallas as pl
from jax.experimental.pallas import tpu as pltpu
from jax.experimental.pallas import tpu_sc as plsc

N = 50000
EMB = 64
HW = 32
NP = 51200
NPAD = 51200
TRASH = 51100
NTILES = 16
RPT = NPAD // NTILES
CH = 128
GROUP = 8
NGROUPS = 49
CPT = GROUP * NGROUPS
E_PAD = NTILES * CPT * CH
EHALF = E_PAD // 2
G8 = 7
NG8 = EHALF // (NTILES * CH * G8)
BLK = 3200

_mesh = plsc.VectorSubcoreMesh(core_axis_name="c", subcore_axis_name="s")


def _sc_segsum64(table, srcidx, dst2d, zrows):
    @functools.partial(
        pl.kernel,
        mesh=_mesh,
        out_type=jax.ShapeDtypeStruct((NPAD, 128), jnp.float32),
        compiler_params=pltpu.CompilerParams(use_tc_tiling_on_sc=False, needs_layout_passes=False),
        scratch_types=[
            pltpu.VMEM_SHARED((NPAD, HW), jnp.float32),
            pltpu.VMEM((GROUP * CH,), jnp.int32),
            pltpu.VMEM((GROUP, CH), jnp.int32),
            pltpu.VMEM((4, CH, HW), jnp.float32),
            [pltpu.SemaphoreType.DMA] * 4,
            [pltpu.SemaphoreType.DMA] * 2,
        ],
    )
    def k(tab_hbm, src_hbm, dst_hbm, z_hbm, out_hbm,
          acc, gsrc, gdst, rows, gsems, ssems):
        c = lax.axis_index("c")
        t = lax.axis_index("s")

        pltpu.sync_copy(z_hbm, acc.at[pl.ds(t * RPT, RPT)])
        plsc.subcore_barrier()

        tile_edge = t * CPT * CH
        tile_drow = t * CPT

        def gather(j):
            return pltpu.async_copy(
                tab_hbm.at[gsrc.at[pl.ds(j * CH, CH)]],
                rows.at[j % 4], gsems[j % 4])

        def group_body(g, carry):
            eoff = tile_edge + g * (GROUP * CH)
            pltpu.sync_copy(src_hbm.at[pl.ds(c * E_PAD + eoff, GROUP * CH)],
                            gsrc)
            pltpu.sync_copy(dst_hbm.at[pl.ds(tile_drow + g * GROUP, GROUP)],
                            gdst)
            gs = [gather(0), gather(1)]
            ss = [None] * GROUP
            for j in range(GROUP):
                if j >= 2:
                    ss[j - 2].wait()
                if j + 2 < GROUP:
                    gs.append(gather(j + 2))
                gs[j].wait()
                ss[j] = pltpu.async_copy(
                    rows.at[j % 4], acc.at[gdst.at[j]],
                    ssems[j % 2], add=True)
            ss[GROUP - 2].wait()
            ss[GROUP - 1].wait()
            return carry

        lax.fori_loop(0, NGROUPS, group_body, 0)
        plsc.subcore_barrier()
        pltpu.sync_copy(acc.at[pl.ds(t * RPT, RPT)],
                        out_hbm.at[pl.ds(t * RPT, RPT), pl.ds(c * HW, HW)])

    return k(table, srcidx, dst2d, zrows)


def _sc_segsum8(c0, c1, c2, c3, dst2d, zrows):
    @functools.partial(
        pl.kernel,
        mesh=_mesh,
        out_type=jax.ShapeDtypeStruct((NPAD, 128), jnp.float32),
        compiler_params=pltpu.CompilerParams(use_tc_tiling_on_sc=False, needs_layout_passes=False),
        scratch_types=[
            pltpu.VMEM_SHARED((NPAD, 8), jnp.float32),
            pltpu.VMEM((4, G8 * CH), jnp.float32),
            pltpu.VMEM((G8, CH), jnp.int32),
            pltpu.VMEM((CH, 8), jnp.float32),
        ],
    )
    def k(c0_hbm, c1_hbm, c2_hbm, c3_hbm, dst_hbm, z_hbm, out_hbm,
          acc, cols, gdst, rows8):
        c = lax.axis_index("c")
        t = lax.axis_index("s")

        pltpu.sync_copy(z_hbm, acc.at[pl.ds(t * RPT, RPT)])

        lanes = lax.iota(jnp.int32, 16)

        def init_body(kk, carry):
            f = kk * 16 + lanes
            v = jnp.where((f & 7) == 4, 1.0, 0.0).astype(jnp.float32)
            plsc.store_scatter(rows8, [f >> 3, f & 7], v)
            return carry

        lax.fori_loop(0, CH * 8 // 16, init_body, 0)
        plsc.subcore_barrier()

        tile_edge = c * EHALF + t * NG8 * G8 * CH
        tile_drow = tile_edge // CH
        chbms = [c0_hbm, c1_hbm, c2_hbm, c3_hbm]

        def group_body(g, carry):
            eoff = tile_edge + g * (G8 * CH)
            for ci in range(4):
                pltpu.sync_copy(chbms[ci].at[pl.ds(eoff, G8 * CH)],
                                cols.at[ci])
            pltpu.sync_copy(dst_hbm.at[pl.ds(tile_drow + g * G8, G8)], gdst)
            for j in range(G8):
                def ileave(q, carry2, _j=j):
                    e = q * 16 + lanes
                    for ci in range(4):
                        v = cols[ci, pl.ds(_j * CH + q * 16, 16)]
                        plsc.store_scatter(
                            rows8, [e, jnp.full((16,), ci, jnp.int32)], v)
                    return carry2
                lax.fori_loop(0, CH // 16, ileave, 0)
                pltpu.sync_copy(rows8, acc.at[gdst.at[j]], add=True)
            return carry

        lax.fori_loop(0, NG8, group_body, 0)
        plsc.subcore_barrier()
        pltpu.sync_copy(acc.at[pl.ds(t * RPT, RPT)],
                        out_hbm.at[pl.ds(t * RPT, RPT), pl.ds(c * 8, 8)])

    return k(c0, c1, c2, c3, dst2d, zrows)


def _mm(a, b):
    return jnp.dot(a, b, preferred_element_type=jnp.float32)


_W128 = lambda i: (i, 0)
_W0 = lambda i: (0, 0)


def _embed(x8, Wn8, bn8):
    def body(x_ref, w_ref, b_ref, o_ref):
        r = jnp.maximum(_mm(x_ref[...], w_ref[...]) + b_ref[0:1, :], 0.0)
        o_ref[...] = jnp.concatenate(
            [r, jnp.zeros((BLK, 128 - EMB), jnp.float32)], axis=1)

    return pl.pallas_call(
        body,
        grid=(NP // BLK,),
        in_specs=[pl.BlockSpec((BLK, 8), _W128),
                  pl.BlockSpec((8, EMB), _W0),
                  pl.BlockSpec((8, EMB), _W0)],
        out_specs=pl.BlockSpec((BLK, 128), _W128),
        out_shape=jax.ShapeDtypeStruct((NP, 128), jnp.float32),
    )(x8, Wn8, bn8)


def _update(h128, p128, s128, Wed8, We_i, be_i8, Wu_i, bu_i8):
    def body(h_ref, p_ref, s_ref, wed_ref, we_ref, be_ref, wu_ref, bu_ref,
             o_ref):
        sc8 = s_ref[:, 0:8] + s_ref[:, 8:16]
        cnt = sc8[:, 4:5]
        sea = _mm(sc8, wed_ref[...])
        agg = (_mm(p_ref[:, 0:EMB], we_ref[0:EMB, :])
               + _mm(sea, we_ref[EMB:2 * EMB, :])
               + cnt * be_ref[0:1, :]) / jnp.maximum(cnt, 1.0)
        o = (_mm(h_ref[:, 0:EMB], wu_ref[0:EMB, :])
             + _mm(agg, wu_ref[EMB:2 * EMB, :]) + bu_ref[0:1, :])
        o = jnp.maximum(o, 0.0)
        o_ref[...] = jnp.concatenate(
            [o, jnp.zeros((BLK, 128 - EMB), jnp.float32)], axis=1)

    return pl.pallas_call(
        body,
        grid=(NP // BLK,),
        in_specs=[pl.BlockSpec((BLK, 128), _W128),
                  pl.BlockSpec((BLK, 128), _W128),
                  pl.BlockSpec((BLK, 128), _W128),
                  pl.BlockSpec((8, EMB), _W0),
                  pl.BlockSpec((2 * EMB, EMB), _W0),
                  pl.BlockSpec((8, EMB), _W0),
                  pl.BlockSpec((2 * EMB, EMB), _W0),
                  pl.BlockSpec((8, EMB), _W0)],
        out_specs=pl.BlockSpec((BLK, 128), _W128),
        out_shape=jax.ShapeDtypeStruct((NP, 128), jnp.float32),
    )(h128, p128, s128, Wed8, We_i, be_i8, Wu_i, bu_i8)


def _decode(h128, Wd8, bd8):
    def body(h_ref, w_ref, b_ref, o_ref):
        o_ref[...] = _mm(h_ref[:, 0:EMB], w_ref[...]) + b_ref[0:1, :]

    return pl.pallas_call(
        body,
        grid=(NP // BLK,),
        in_specs=[pl.BlockSpec((BLK, 128), _W128),
                  pl.BlockSpec((EMB, 8), _W0),
                  pl.BlockSpec((8, 8), _W0)],
        out_specs=pl.BlockSpec((BLK, 8), _W128),
        out_shape=jax.ShapeDtypeStruct((NP, 8), jnp.float32),
    )(h128, Wd8, bd8)


def kernel(x, edge_attr, edge_index, Wn, bn, Wed, bed, We, be, Wu, bu, Wd, bd):
    E = edge_index.shape[1]
    src = edge_index[0]
    dst = edge_index[1]

    src_pad = jnp.concatenate([src, jnp.zeros((E_PAD - E,), jnp.int32)])
    srcidx = jnp.concatenate([4 * src_pad, 4 * src_pad + 1])
    dst2d = jnp.concatenate(
        [dst, jnp.full((E_PAD - E,), TRASH, jnp.int32)]).reshape(-1, CH)
    zpad = jnp.zeros((E_PAD - E,), jnp.float32)
    ecols = [jnp.concatenate([edge_attr[:, i], zpad]) for i in range(4)]

    x8 = jnp.pad(x, ((0, NP - N), (0, 1)))
    Wn8 = jnp.concatenate([Wn, jnp.zeros((1, EMB), jnp.float32)], axis=0)
    bn8 = jnp.broadcast_to(bn[None, :], (8, EMB))
    Wed8 = jnp.concatenate(
        [Wed, bed[None, :], jnp.zeros((3, EMB), jnp.float32)], axis=0)
    Wd8 = jnp.concatenate([Wd, jnp.zeros((EMB, 5), jnp.float32)], axis=1)
    bd8 = jnp.broadcast_to(
        jnp.concatenate([bd, jnp.zeros((5,), jnp.float32)])[None, :], (8, 8))

    z32 = jnp.zeros((RPT, HW), jnp.float32)
    z8 = jnp.zeros((RPT, 8), jnp.float32)

    s128 = _sc_segsum8(*ecols, dst2d, z8)

    h128 = _embed(x8, Wn8, bn8)
    for i in range(6):
        table = h128.reshape(4 * NP, HW)
        p128 = _sc_segsum64(table, srcidx, dst2d, z32)
        h128 = _update(h128, p128, s128, Wed8, We[i],
                       jnp.broadcast_to(be[i][None, :], (8, EMB)),
                       Wu[i],
                       jnp.broadcast_to(bu[i][None, :], (8, EMB)))
    return _decode(h128, Wd8, bd8)[:N, :3]


# speedup vs baseline: 9.5725x; 1.0479x over previous
import functools

import jax
import jax.numpy as jnp
from jax import lax
from jax.experimental import pallas as pl
from jax.experimental.pallas import tpu as pltpu
from jax.experimental.pallas import tpu_sc as plsc

N = 50000
EMB = 64
HW = 32
NP = 51200
NPAD = 51200
TRASH = 51100
NTILES = 16
RPT = NPAD // NTILES
CH = 128
GROUP = 8
NGROUPS = 49
CPT = GROUP * NGROUPS
E_PAD = NTILES * CPT * CH
EHALF = E_PAD // 2
G8 = 7
NG8 = EHALF // (NTILES * CH * G8)
BLK = 3200

_mesh = plsc.VectorSubcoreMesh(core_axis_name="c", subcore_axis_name="s")


def _sc_segsum64(table, srcidx, dst2d, zrows):
    @functools.partial(
        pl.kernel,
        mesh=_mesh,
        out_type=jax.ShapeDtypeStruct((NPAD, 128), jnp.float32),
        compiler_params=pltpu.CompilerParams(use_tc_tiling_on_sc=False, needs_layout_passes=False),
        scratch_types=[
            pltpu.VMEM_SHARED((NPAD, HW), jnp.float32),
            pltpu.VMEM((GROUP * CH,), jnp.int32),
            pltpu.VMEM((GROUP, CH), jnp.int32),
            pltpu.VMEM((6, CH, HW), jnp.float32),
            [pltpu.SemaphoreType.DMA] * 6,
            [pltpu.SemaphoreType.DMA] * 3,
        ],
    )
    def k(tab_hbm, src_hbm, dst_hbm, z_hbm, out_hbm,
          acc, gsrc, gdst, rows, gsems, ssems):
        c = lax.axis_index("c")
        t = lax.axis_index("s")

        pltpu.sync_copy(z_hbm, acc.at[pl.ds(t * RPT, RPT)])
        plsc.subcore_barrier()

        tile_edge = t * CPT * CH
        tile_drow = t * CPT

        def gather(j):
            return pltpu.async_copy(
                tab_hbm.at[gsrc.at[pl.ds(j * CH, CH)]],
                rows.at[j % 6], gsems[j % 6])

        def group_body(g, carry):
            eoff = tile_edge + g * (GROUP * CH)
            pltpu.sync_copy(src_hbm.at[pl.ds(c * E_PAD + eoff, GROUP * CH)],
                            gsrc)
            pltpu.sync_copy(dst_hbm.at[pl.ds(tile_drow + g * GROUP, GROUP)],
                            gdst)
            gs = [gather(0), gather(1), gather(2)]
            ss = [None] * GROUP
            for j in range(GROUP):
                if j >= 3:
                    ss[j - 3].wait()
                if j + 3 < GROUP:
                    gs.append(gather(j + 3))
                gs[j].wait()
                ss[j] = pltpu.async_copy(
                    rows.at[j % 6], acc.at[gdst.at[j]],
                    ssems[j % 3], add=True)
            for j in range(GROUP - 3, GROUP):
                ss[j].wait()
            return carry

        lax.fori_loop(0, NGROUPS, group_body, 0)
        plsc.subcore_barrier()
        pltpu.sync_copy(acc.at[pl.ds(t * RPT, RPT)],
                        out_hbm.at[pl.ds(t * RPT, RPT), pl.ds(c * HW, HW)])

    return k(table, srcidx, dst2d, zrows)


def _sc_segsum8(c0, c1, c2, c3, dst2d, zrows):
    @functools.partial(
        pl.kernel,
        mesh=_mesh,
        out_type=jax.ShapeDtypeStruct((NPAD, 128), jnp.float32),
        compiler_params=pltpu.CompilerParams(use_tc_tiling_on_sc=False, needs_layout_passes=False),
        scratch_types=[
            pltpu.VMEM_SHARED((NPAD, 8), jnp.float32),
            pltpu.VMEM((4, G8 * CH), jnp.float32),
            pltpu.VMEM((G8, CH), jnp.int32),
            pltpu.VMEM((CH, 8), jnp.float32),
        ],
    )
    def k(c0_hbm, c1_hbm, c2_hbm, c3_hbm, dst_hbm, z_hbm, out_hbm,
          acc, cols, gdst, rows8):
        c = lax.axis_index("c")
        t = lax.axis_index("s")

        pltpu.sync_copy(z_hbm, acc.at[pl.ds(t * RPT, RPT)])

        lanes = lax.iota(jnp.int32, 16)

        def init_body(kk, carry):
            f = kk * 16 + lanes
            v = jnp.where((f & 7) == 4, 1.0, 0.0).astype(jnp.float32)
            plsc.store_scatter(rows8, [f >> 3, f & 7], v)
            return carry

        lax.fori_loop(0, CH * 8 // 16, init_body, 0)
        plsc.subcore_barrier()

        tile_edge = c * EHALF + t * NG8 * G8 * CH
        tile_drow = tile_edge // CH
        chbms = [c0_hbm, c1_hbm, c2_hbm, c3_hbm]

        def group_body(g, carry):
            eoff = tile_edge + g * (G8 * CH)
            for ci in range(4):
                pltpu.sync_copy(chbms[ci].at[pl.ds(eoff, G8 * CH)],
                                cols.at[ci])
            pltpu.sync_copy(dst_hbm.at[pl.ds(tile_drow + g * G8, G8)], gdst)
            for j in range(G8):
                def ileave(q, carry2, _j=j):
                    e = q * 16 + lanes
                    for ci in range(4):
                        v = cols[ci, pl.ds(_j * CH + q * 16, 16)]
                        plsc.store_scatter(
                            rows8, [e, jnp.full((16,), ci, jnp.int32)], v)
                    return carry2
                lax.fori_loop(0, CH // 16, ileave, 0)
                pltpu.sync_copy(rows8, acc.at[gdst.at[j]], add=True)
            return carry

        lax.fori_loop(0, NG8, group_body, 0)
        plsc.subcore_barrier()
        pltpu.sync_copy(acc.at[pl.ds(t * RPT, RPT)],
                        out_hbm.at[pl.ds(t * RPT, RPT), pl.ds(c * 8, 8)])

    return k(c0, c1, c2, c3, dst2d, zrows)


def _mm(a, b):
    return jnp.dot(a, b, preferred_element_type=jnp.float32)


_W128 = lambda i: (i, 0)
_W0 = lambda i: (0, 0)


def _embed(x8, Wn8, bn8):
    def body(x_ref, w_ref, b_ref, o_ref):
        r = jnp.maximum(_mm(x_ref[...], w_ref[...]) + b_ref[0:1, :], 0.0)
        o_ref[...] = jnp.concatenate(
            [r, jnp.zeros((BLK, 128 - EMB), jnp.float32)], axis=1)

    return pl.pallas_call(
        body,
        grid=(NP // BLK,),
        in_specs=[pl.BlockSpec((BLK, 8), _W128),
                  pl.BlockSpec((8, EMB), _W0),
                  pl.BlockSpec((8, EMB), _W0)],
        out_specs=pl.BlockSpec((BLK, 128), _W128),
        out_shape=jax.ShapeDtypeStruct((NP, 128), jnp.float32),
    )(x8, Wn8, bn8)


def _update(h128, p128, s128, Wed8, We_i, be_i8, Wu_i, bu_i8):
    def body(h_ref, p_ref, s_ref, wed_ref, we_ref, be_ref, wu_ref, bu_ref,
             o_ref):
        sc8 = s_ref[:, 0:8] + s_ref[:, 8:16]
        cnt = sc8[:, 4:5]
        sea = _mm(sc8, wed_ref[...])
        agg = (_mm(p_ref[:, 0:EMB], we_ref[0:EMB, :])
               + _mm(sea, we_ref[EMB:2 * EMB, :])
               + cnt * be_ref[0:1, :]) / jnp.maximum(cnt, 1.0)
        o = (_mm(h_ref[:, 0:EMB], wu_ref[0:EMB, :])
             + _mm(agg, wu_ref[EMB:2 * EMB, :]) + bu_ref[0:1, :])
        o = jnp.maximum(o, 0.0)
        o_ref[...] = jnp.concatenate(
            [o, jnp.zeros((BLK, 128 - EMB), jnp.float32)], axis=1)

    return pl.pallas_call(
        body,
        grid=(NP // BLK,),
        in_specs=[pl.BlockSpec((BLK, 128), _W128),
                  pl.BlockSpec((BLK, 128), _W128),
                  pl.BlockSpec((BLK, 128), _W128),
                  pl.BlockSpec((8, EMB), _W0),
                  pl.BlockSpec((2 * EMB, EMB), _W0),
                  pl.BlockSpec((8, EMB), _W0),
                  pl.BlockSpec((2 * EMB, EMB), _W0),
                  pl.BlockSpec((8, EMB), _W0)],
        out_specs=pl.BlockSpec((BLK, 128), _W128),
        out_shape=jax.ShapeDtypeStruct((NP, 128), jnp.float32),
    )(h128, p128, s128, Wed8, We_i, be_i8, Wu_i, bu_i8)


def _decode(h128, Wd8, bd8):
    def body(h_ref, w_ref, b_ref, o_ref):
        o_ref[...] = _mm(h_ref[:, 0:EMB], w_ref[...]) + b_ref[0:1, :]

    return pl.pallas_call(
        body,
        grid=(NP // BLK,),
        in_specs=[pl.BlockSpec((BLK, 128), _W128),
                  pl.BlockSpec((EMB, 8), _W0),
                  pl.BlockSpec((8, 8), _W0)],
        out_specs=pl.BlockSpec((BLK, 8), _W128),
        out_shape=jax.ShapeDtypeStruct((NP, 8), jnp.float32),
    )(h128, Wd8, bd8)


def kernel(x, edge_attr, edge_index, Wn, bn, Wed, bed, We, be, Wu, bu, Wd, bd):
    E = edge_index.shape[1]
    src = edge_index[0]
    dst = edge_index[1]

    src_pad = jnp.concatenate([src, jnp.zeros((E_PAD - E,), jnp.int32)])
    srcidx = jnp.concatenate([4 * src_pad, 4 * src_pad + 1])
    dst2d = jnp.concatenate(
        [dst, jnp.full((E_PAD - E,), TRASH, jnp.int32)]).reshape(-1, CH)
    zpad = jnp.zeros((E_PAD - E,), jnp.float32)
    ecols = [jnp.concatenate([edge_attr[:, i], zpad]) for i in range(4)]

    x8 = jnp.pad(x, ((0, NP - N), (0, 1)))
    Wn8 = jnp.concatenate([Wn, jnp.zeros((1, EMB), jnp.float32)], axis=0)
    bn8 = jnp.broadcast_to(bn[None, :], (8, EMB))
    Wed8 = jnp.concatenate(
        [Wed, bed[None, :], jnp.zeros((3, EMB), jnp.float32)], axis=0)
    Wd8 = jnp.concatenate([Wd, jnp.zeros((EMB, 5), jnp.float32)], axis=1)
    bd8 = jnp.broadcast_to(
        jnp.concatenate([bd, jnp.zeros((5,), jnp.float32)])[None, :], (8, 8))

    z32 = jnp.zeros((RPT, HW), jnp.float32)
    z8 = jnp.zeros((RPT, 8), jnp.float32)

    s128 = _sc_segsum8(*ecols, dst2d, z8)

    h128 = _embed(x8, Wn8, bn8)
    for i in range(6):
        table = h128.reshape(4 * NP, HW)
        p128 = _sc_segsum64(table, srcidx, dst2d, z32)
        h128 = _update(h128, p128, s128, Wed8, We[i],
                       jnp.broadcast_to(be[i][None, :], (8, EMB)),
                       Wu[i],
                       jnp.broadcast_to(bu[i][None, :], (8, EMB)))
    return _decode(h128, Wd8, bd8)[:N, :3]
